# baseline (speedup 1.0000x reference)
import jax
import jax.numpy as jnp
from jax import lax
from jax.experimental import pallas as pl
from jax.experimental.pallas import tpu as pltpu
from jax.experimental.pallas import tpu_sc as plsc
import dataclasses
import functools

_sc_params = pltpu.CompilerParams()
if "needs_layout_passes" in pltpu.CompilerParams.__dataclass_fields__:
    _sc_params = dataclasses.replace(_sc_params, needs_layout_passes=False)

N = 524288
D = 128
H = 21
TASK = 3

NC = 2
NS = 16
NW = NC * NS
CHUNK = N // NW
KG = 128
BT = 2048
NBLK = N // BT
BPW = CHUNK // BT

_mesh = plsc.VectorSubcoreMesh(core_axis_name="c", subcore_axis_name="s")


def _wid():
    return lax.axis_index("s") * NC + lax.axis_index("c")


@functools.partial(
    pl.kernel,
    out_type=[
        jax.ShapeDtypeStruct((N, D), jnp.float32),
        jax.ShapeDtypeStruct((NW, CHUNK), jnp.int32),
        jax.ShapeDtypeStruct((NW, 16), jnp.int32),
    ],
    mesh=_mesh,
    scratch_types=[
        pltpu.VMEM((CHUNK,), jnp.int32),
        pltpu.VMEM((CHUNK + KG,), jnp.int32),
        pltpu.VMEM((KG, D), jnp.float32),
        pltpu.VMEM((16,), jnp.int32),
        pltpu.SemaphoreType.DMA,
    ],
    compiler_params=_sc_params,
)
def _sc_compact_gather(x1_hbm, x0_hbm, xg_hbm, idx_hbm, cnt_hbm,
                       x1_v, idx_v, rows_v, cnt_v, sem):
    w = _wid()
    base = w * CHUNK
    pltpu.sync_copy(x1_hbm.at[pl.ds(base, CHUNK)], x1_v)

    lanes = lax.iota(jnp.int32, 16)

    @pl.loop(0, CHUNK // 16, init_carry=jnp.int32(0), unroll=4)
    def cnt(i, c):
        v = x1_v[pl.ds(i * 16, 16)]
        m = v == TASK
        gi = (base + i * 16) + lanes
        plsc.store_compressed(idx_v.at[pl.ds(c, 16)], gi, mask=m)
        return c + jnp.sum(m.astype(jnp.int32))

    safe = jnp.full((16,), base, jnp.int32)
    for p in range(KG // 16):
        idx_v[pl.ds(cnt + p * 16, 16)] = safe

    cnt_v[...] = jnp.full((16,), cnt, jnp.int32)
    pltpu.sync_copy(cnt_v, cnt_hbm.at[w])
    pltpu.sync_copy(idx_v.at[pl.ds(0, CHUNK)], idx_hbm.at[w])

    nch = (cnt + (KG - 1)) // KG

    @pl.loop(0, nch)
    def _(j):
        off = j * KG
        pltpu.async_copy(x0_hbm.at[idx_v.at[pl.ds(off, KG)]], rows_v, sem).wait()
        pltpu.sync_copy(rows_v, xg_hbm.at[pl.ds(base + off, KG)])


def _mlp_body(tbl_ref, xg_ref, w1_ref, b1_ref, w2_ref, b2_ref, h_ref):
    i = pl.program_id(0)

    @pl.when(i < tbl_ref[0])
    def _():
        x = xg_ref[...]
        z = lax.dot_general(
            w1_ref[...], x, (((0,), (1,)), ((), ())),
            preferred_element_type=jnp.float32,
        )
        z = z + b1_ref[...]
        g = 0.5 * z * (1.0 + lax.erf(z * 0.7071067811865476))
        h_ref[0] = jnp.sum(g * w2_ref[...], axis=0, keepdims=True) + b2_ref[...]


def _tc_mlp(tbl, xg, W1, b1, W2, b2):
    return pl.pallas_call(
        _mlp_body,
        grid_spec=pltpu.PrefetchScalarGridSpec(
            num_scalar_prefetch=1,
            grid=(NBLK,),
            in_specs=[
                pl.BlockSpec((BT, D), lambda i, tbl: (tbl[1 + i], 0)),
                pl.BlockSpec((D, H), lambda i, tbl: (0, 0)),
                pl.BlockSpec((H, 1), lambda i, tbl: (0, 0)),
                pl.BlockSpec((H, 1), lambda i, tbl: (0, 0)),
                pl.BlockSpec((1, 1), lambda i, tbl: (0, 0)),
            ],
            out_specs=pl.BlockSpec((1, 1, BT), lambda i, tbl: (tbl[1 + i], 0, 0)),
        ),
        out_shape=jax.ShapeDtypeStruct((NBLK, 1, BT), jnp.float32),
    )(tbl, xg, W1, b1.reshape(H, 1), W2, b2.reshape(1, 1))


@functools.partial(
    pl.kernel,
    out_type=jax.ShapeDtypeStruct((N,), jnp.float32),
    mesh=_mesh,
    scratch_types=[
        pltpu.VMEM((CHUNK,), jnp.int32),
        pltpu.VMEM((CHUNK,), jnp.float32),
        pltpu.VMEM((CHUNK,), jnp.float32),
        pltpu.VMEM((16,), jnp.int32),
    ],
    compiler_params=_sc_params,
)
def _sc_scatter(idx_hbm, cnt_hbm, h_hbm, x2_hbm, out_hbm,
                idx_v, h_v, out_v, cnt_v):
    w = _wid()
    base = w * CHUNK
    pltpu.sync_copy(cnt_hbm.at[w], cnt_v)
    pltpu.sync_copy(idx_hbm.at[w], idx_v)
    pltpu.sync_copy(h_hbm.at[pl.ds(base, CHUNK)], h_v)
    pltpu.sync_copy(x2_hbm.at[pl.ds(base, CHUNK)], out_v)
    cnt = cnt_v[...][0]
    lanes = lax.iota(jnp.int32, 16)

    @pl.loop(0, (cnt + 15) // 16)
    def _(k):
        off = k * 16
        iv = idx_v[pl.ds(off, 16)] - base
        hv = h_v[pl.ds(off, 16)]
        m = (off + lanes) < cnt
        plsc.store_scatter(out_v, [iv], hv, mask=m)

    pltpu.sync_copy(out_v, out_hbm.at[pl.ds(base, CHUNK)])


def kernel(x0, x1, x2, W1, b1, W2, b2):
    x1i = x1.astype(jnp.int32).reshape(N)
    xg, idxs, cnt16 = _sc_compact_gather(x1i, x0)
    cnt = cnt16[:, 0]

    nb = (cnt + (BT - 1)) // BT
    ends = jnp.cumsum(nb)
    starts = ends - nb
    total = ends[-1]
    t = jnp.arange(NBLK, dtype=jnp.int32)
    w_of_t = jnp.searchsorted(ends, t, side="right").astype(jnp.int32)
    w_cl = jnp.minimum(w_of_t, NW - 1)
    blk = w_cl * BPW + (t - starts[w_cl])
    last = blk[jnp.maximum(total - 1, 0)]
    tbl_body = jnp.where(t < total, blk, last).astype(jnp.int32)
    tbl = jnp.concatenate([total[None].astype(jnp.int32), tbl_body])

    h = _tc_mlp(tbl, xg, W1, b1, W2, b2)
    x2_new = _sc_scatter(idxs, cnt16, h.reshape(N), x2.reshape(N))
    return (x0, x1, x2_new.reshape(N, 1))


# speedup vs baseline: 1.4452x; 1.4452x over previous
import jax
import jax.numpy as jnp
from jax import lax
from jax.experimental import pallas as pl
from jax.experimental.pallas import tpu as pltpu
from jax.experimental.pallas import tpu_sc as plsc
import dataclasses
import functools

_sc_params = pltpu.CompilerParams()
if "needs_layout_passes" in pltpu.CompilerParams.__dataclass_fields__:
    _sc_params = dataclasses.replace(_sc_params, needs_layout_passes=False)

N = 524288
D = 128
H = 21
TASK = 3

NC = 2
NS = 16
NW = NC * NS
CHUNK = N // NW
KG = 128
BT = 4096
NBLK = N // BT
BPW = CHUNK // BT

_mesh = plsc.VectorSubcoreMesh(core_axis_name="c", subcore_axis_name="s")


def _wid():
    return lax.axis_index("s") * NC + lax.axis_index("c")


@functools.partial(
    pl.kernel,
    out_type=[
        jax.ShapeDtypeStruct((N, D), jnp.float32),
        jax.ShapeDtypeStruct((NW, CHUNK), jnp.int32),
        jax.ShapeDtypeStruct((NW, 16), jnp.int32),
    ],
    mesh=_mesh,
    scratch_types=[
        pltpu.VMEM((CHUNK,), jnp.int32),
        pltpu.VMEM((CHUNK + KG,), jnp.int32),
        pltpu.VMEM((KG, D), jnp.float32),
        pltpu.VMEM((16,), jnp.int32),
        pltpu.SemaphoreType.DMA,
    ],
    compiler_params=_sc_params,
)
def _sc_compact_gather(x1_hbm, x0_hbm, xg_hbm, idx_hbm, cnt_hbm,
                       x1_v, idx_v, rows_v, cnt_v, sem):
    w = _wid()
    base = w * CHUNK
    pltpu.sync_copy(x1_hbm.at[pl.ds(base, CHUNK)], x1_v)

    lanes = lax.iota(jnp.int32, 16)

    @pl.loop(0, CHUNK // 16, init_carry=jnp.int32(0), unroll=4)
    def cnt(i, c):
        v = x1_v[pl.ds(i * 16, 16)]
        m = v == TASK
        gi = (base + i * 16) + lanes
        plsc.store_compressed(idx_v.at[pl.ds(c, 16)], gi, mask=m)
        return c + jnp.sum(m.astype(jnp.int32))

    safe = jnp.full((16,), base, jnp.int32)
    for p in range(KG // 16):
        idx_v[pl.ds(cnt + p * 16, 16)] = safe

    cnt_v[...] = jnp.full((16,), cnt, jnp.int32)
    pltpu.sync_copy(cnt_v, cnt_hbm.at[w])
    pltpu.sync_copy(idx_v.at[pl.ds(0, CHUNK)], idx_hbm.at[w])

    nch = (cnt + (KG - 1)) // KG

    @pl.loop(0, nch)
    def _(j):
        off = j * KG
        pltpu.async_copy(x0_hbm.at[idx_v.at[pl.ds(off, KG)]], rows_v, sem).wait()
        pltpu.sync_copy(rows_v, xg_hbm.at[pl.ds(base + off, KG)])


def _mlp_body(cnt_ref, xg_ref, w1_ref, b1_ref, w2_ref, b2_ref, h_ref):
    w = pl.program_id(0)
    j = pl.program_id(1)

    @pl.when(j * BT < cnt_ref[w, 0])
    def _():
        x = xg_ref[...]
        z = lax.dot_general(
            w1_ref[...], x, (((0,), (1,)), ((), ())),
            preferred_element_type=jnp.float32,
        )
        z = z + b1_ref[...]
        g = 0.5 * z * (1.0 + lax.erf(z * 0.7071067811865476))
        h_ref[0] = jnp.sum(g * w2_ref[...], axis=0, keepdims=True) + b2_ref[...]


def _xg_map(w, j, cnt_ref):
    nb = (cnt_ref[w, 0] + (BT - 1)) // BT
    jc = jnp.minimum(j, jnp.maximum(nb - 1, 0))
    return (w * BPW + jc, 0)


def _h_map(w, j, cnt_ref):
    b, _ = _xg_map(w, j, cnt_ref)
    return (b, 0, 0)


def _tc_mlp(cnt16, xg, W1, b1, W2, b2):
    return pl.pallas_call(
        _mlp_body,
        grid_spec=pltpu.PrefetchScalarGridSpec(
            num_scalar_prefetch=1,
            grid=(NW, BPW),
            in_specs=[
                pl.BlockSpec((BT, D), _xg_map),
                pl.BlockSpec((D, H), lambda w, j, c: (0, 0)),
                pl.BlockSpec((H, 1), lambda w, j, c: (0, 0)),
                pl.BlockSpec((H, 1), lambda w, j, c: (0, 0)),
                pl.BlockSpec((1, 1), lambda w, j, c: (0, 0)),
            ],
            out_specs=pl.BlockSpec((1, 1, BT), _h_map),
        ),
        out_shape=jax.ShapeDtypeStruct((NBLK, 1, BT), jnp.float32),
    )(cnt16, xg, W1, b1.reshape(H, 1), W2, b2.reshape(1, 1))


@functools.partial(
    pl.kernel,
    out_type=jax.ShapeDtypeStruct((N,), jnp.float32),
    mesh=_mesh,
    scratch_types=[
        pltpu.VMEM((CHUNK,), jnp.int32),
        pltpu.VMEM((CHUNK,), jnp.float32),
        pltpu.VMEM((CHUNK,), jnp.float32),
        pltpu.VMEM((16,), jnp.int32),
    ],
    compiler_params=_sc_params,
)
def _sc_scatter(idx_hbm, cnt_hbm, h_hbm, x2_hbm, out_hbm,
                idx_v, h_v, out_v, cnt_v):
    w = _wid()
    base = w * CHUNK
    pltpu.sync_copy(cnt_hbm.at[w], cnt_v)
    pltpu.sync_copy(idx_hbm.at[w], idx_v)
    pltpu.sync_copy(h_hbm.at[pl.ds(base, CHUNK)], h_v)
    pltpu.sync_copy(x2_hbm.at[pl.ds(base, CHUNK)], out_v)
    cnt = cnt_v[...][0]
    lanes = lax.iota(jnp.int32, 16)

    @pl.loop(0, (cnt + 15) // 16)
    def _(k):
        off = k * 16
        iv = idx_v[pl.ds(off, 16)] - base
        hv = h_v[pl.ds(off, 16)]
        m = (off + lanes) < cnt
        plsc.store_scatter(out_v, [iv], hv, mask=m)

    pltpu.sync_copy(out_v, out_hbm.at[pl.ds(base, CHUNK)])


def kernel(x0, x1, x2, W1, b1, W2, b2):
    x0_out = jnp.copy(x0)
    x1i = x1.astype(jnp.int32).reshape(N)
    xg, idxs, cnt16 = _sc_compact_gather(x1i, x0)
    h = _tc_mlp(cnt16, xg, W1, b1, W2, b2)
    x2_new = _sc_scatter(idxs, cnt16, h.reshape(N), x2.reshape(N))
    return (x0_out, x1, x2_new.reshape(N, 1))
